# baseline (speedup 1.0000x reference)
import functools

import jax
import jax.numpy as jnp
from jax import lax
from jax.experimental import pallas as pl
from jax.experimental.pallas import tpu as pltpu
from jax.experimental.pallas import tpu_sc as plsc

_CB = 8192
_D = 32
_TB = 1024
_CH = 2048
_NCH = _CB // _CH
_BETA = 0.25

_NC = 2
_NS = 16
_NW = _NC * _NS
_BPW = _CB // _NW
_IDXC = 128


def _argmin_body(cn_ref, zn_ref, cb_ref, zb_ref, idx_ref):
    zbv = zb_ref[...]
    rid = lax.broadcasted_iota(jnp.int32, (_CH, _TB), 0)

    def chunk(j, carry):
        bv, bi = carry
        cbc = cb_ref[pl.ds(j * _CH, _CH), :]
        mm = lax.dot_general(
            cbc, zbv, (((1,), (1,)), ((), ())),
            preferred_element_type=jnp.float32)
        t = cn_ref[pl.ds(j * _CH, _CH), :] + zn_ref[0]
        d = t - 2.0 * mm
        m = jnp.min(d, axis=0, keepdims=True)
        cand = jnp.where(d == m, rid, _CB)
        ci = jnp.min(cand, axis=0, keepdims=True) + j * _CH
        keep = bv <= m
        nv = jnp.where(keep, bv, m)
        ni = jnp.where(keep, bi, ci)
        nv = nv.astype(jnp.bfloat16).astype(jnp.float32)
        return nv, ni

    bv0 = jnp.full((1, _TB), jnp.inf, jnp.float32)
    bi0 = jnp.zeros((1, _TB), jnp.int32)
    _, bi = lax.fori_loop(0, _NCH, chunk, (bv0, bi0))
    idx_ref[...] = bi.reshape(1, 1, _TB)


def _argmin_call(cn2, zn3, cbb, zb):
    return pl.pallas_call(
        _argmin_body,
        grid=(8,),
        in_specs=[
            pl.BlockSpec((_CB, 1), lambda i: (0, 0)),
            pl.BlockSpec((1, 1, _TB), lambda i: (i, 0, 0)),
            pl.BlockSpec((_CB, _D), lambda i: (0, 0)),
            pl.BlockSpec((_TB, _D), lambda i: (i, 0)),
        ],
        out_specs=pl.BlockSpec((1, 1, _TB), lambda i: (i, 0, 0)),
        out_shape=jax.ShapeDtypeStruct((8, 1, _TB), jnp.int32),
    )(cn2, zn3, cbb, zb)


def _sc_body(cb_hbm, idx_hbm, zeros_hbm, ones_hbm, zq_hbm, cnt_hbm,
             idx_v0, idx_v1, rows_v0, rows_v1, ones_v, slab_v, counts_sp,
             sem0, sem1):
    c = lax.axis_index("c")
    s = lax.axis_index("s")
    wid = s * _NC + c
    base = wid * _BPW

    pltpu.sync_copy(idx_hbm.at[pl.ds(base, _IDXC)], idx_v0)
    pltpu.sync_copy(idx_hbm.at[pl.ds(base + _IDXC, _IDXC)], idx_v1)
    cp0 = pltpu.async_copy(cb_hbm.at[idx_v0], rows_v0, sem0)
    cp1 = pltpu.async_copy(cb_hbm.at[idx_v1], rows_v1, sem1)

    @pl.when(s == 0)
    def _zero():
        pltpu.sync_copy(zeros_hbm, counts_sp)

    pltpu.sync_copy(ones_hbm, ones_v)
    cp0.wait()
    cp1.wait()
    pltpu.sync_copy(rows_v0, zq_hbm.at[pl.ds(base, _IDXC)])
    pltpu.sync_copy(rows_v1, zq_hbm.at[pl.ds(base + _IDXC, _IDXC)])

    plsc.subcore_barrier()
    pltpu.sync_copy(ones_v, counts_sp.at[idx_v0], add=True)
    pltpu.sync_copy(ones_v, counts_sp.at[idx_v1], add=True)
    plsc.subcore_barrier()

    rows_per_tile = _CB // _NS
    pltpu.sync_copy(counts_sp.at[pl.ds(s * rows_per_tile, rows_per_tile)],
                    slab_v)
    pltpu.sync_copy(slab_v, cnt_hbm.at[c, pl.ds(s * rows_per_tile,
                                                rows_per_tile)])


@functools.partial(
    pl.kernel,
    out_type=(jax.ShapeDtypeStruct((_CB, _D), jnp.float32),
              jax.ShapeDtypeStruct((_NC, _CB, 16), jnp.float32)),
    mesh=plsc.VectorSubcoreMesh(core_axis_name="c", subcore_axis_name="s",
                                num_cores=_NC, num_subcores=_NS),
    compiler_params=pltpu.CompilerParams(use_tc_tiling_on_sc=False),
    scratch_types=(
        pltpu.VMEM((_IDXC,), jnp.int32),
        pltpu.VMEM((_IDXC,), jnp.int32),
        pltpu.VMEM((_IDXC, _D), jnp.float32),
        pltpu.VMEM((_IDXC, _D), jnp.float32),
        pltpu.VMEM((_IDXC, 16), jnp.float32),
        pltpu.VMEM((_CB // _NS, 16), jnp.float32),
        pltpu.VMEM_SHARED((_CB, 16), jnp.float32),
        pltpu.SemaphoreType.DMA,
        pltpu.SemaphoreType.DMA,
    ),
)
def _sc_gather_hist(cb_hbm, idx_hbm, zeros_hbm, ones_hbm, zq_hbm, cnt_hbm,
                    idx_v0, idx_v1, rows_v0, rows_v1, ones_v, slab_v,
                    counts_sp, sem0, sem1):
    _sc_body(cb_hbm, idx_hbm, zeros_hbm, ones_hbm, zq_hbm, cnt_hbm,
             idx_v0, idx_v1, rows_v0, rows_v1, ones_v, slab_v, counts_sp,
             sem0, sem1)


def _epi_body(z_ref, zq_ref, cnt_ref, zqst_ref, loss_ref, perp_ref):
    zv = z_ref[...]
    qv = zq_ref[...]
    diff = qv - zv
    zqst_ref[...] = zv + diff
    msq = jnp.mean(diff * diff)
    loss_ref[...] = jnp.reshape(_BETA * msq + msq, (1, 1))
    c2 = cnt_ref[0] + cnt_ref[1]
    e = c2 * (1.0 / _CB)
    el = e * jnp.log(e + 1e-8)
    ent = jnp.sum(el[:, 0:1])
    perp_ref[...] = jnp.reshape(jnp.exp(-ent), (1, 1))


def _epi_call(z_flat, zq, cnt):
    return pl.pallas_call(
        _epi_body,
        in_specs=[
            pl.BlockSpec((_CB, _D), lambda: (0, 0)),
            pl.BlockSpec((_CB, _D), lambda: (0, 0)),
            pl.BlockSpec((_NC, _CB, 16), lambda: (0, 0, 0)),
        ],
        out_specs=[
            pl.BlockSpec((_CB, _D), lambda: (0, 0)),
            pl.BlockSpec((1, 1), lambda: (0, 0)),
            pl.BlockSpec((1, 1), lambda: (0, 0)),
        ],
        out_shape=[
            jax.ShapeDtypeStruct((_CB, _D), jnp.float32),
            jax.ShapeDtypeStruct((1, 1), jnp.float32),
            jax.ShapeDtypeStruct((1, 1), jnp.float32),
        ],
    )(z_flat, zq, cnt)


def kernel(z, codebook):
    z_flat = z.reshape(-1, _D)
    zb = z_flat.astype(jnp.bfloat16)
    cbb = codebook.astype(jnp.bfloat16)
    zn = jnp.sum(z_flat ** 2, axis=1)
    cn = jnp.sum(codebook ** 2, axis=1)

    idx = _argmin_call(cn.reshape(_CB, 1), zn.reshape(8, 1, _TB),
                       cbb, zb).reshape(-1)

    zeros = jnp.zeros((_CB, 16), jnp.float32)
    ones = jnp.ones((_IDXC, 16), jnp.float32)
    z_q, cnt = _sc_gather_hist(codebook, idx, zeros, ones)

    zq_st, loss, perp = _epi_call(z_flat, z_q, cnt)
    return (zq_st.reshape(z.shape), loss[0, 0], perp[0, 0])


# speedup vs baseline: 1.0421x; 1.0421x over previous
import functools

import jax
import jax.numpy as jnp
from jax import lax
from jax.experimental import pallas as pl
from jax.experimental.pallas import tpu as pltpu
from jax.experimental.pallas import tpu_sc as plsc

_CB = 8192
_D = 32
_TB = 1024
_CH = 2048
_NCH = _CB // _CH
_BETA = 0.25

_NC = 2
_NS = 16
_NW = _NC * _NS
_BPW = _CB // _NW
_IDXC = 128


def _argmin_body(cn_ref, zn_ref, cb2_ref, zb_ref, idx_ref):
    zbv = zb_ref[...]
    rid = lax.broadcasted_iota(jnp.int32, (_CH, _TB), 0).astype(jnp.float32)

    def chunk(j, carry):
        bv, bi = carry
        cbc = cb2_ref[pl.ds(j * _CH, _CH), :]
        mm2 = lax.dot_general(
            cbc, zbv, (((1,), (1,)), ((), ())),
            preferred_element_type=jnp.float32)
        t = cn_ref[pl.ds(j * _CH, _CH), :] + zn_ref[0]
        d = t - mm2
        m = jnp.min(d, axis=0, keepdims=True)
        cand = jnp.where(d == m, rid, jnp.inf)
        cif = jnp.min(cand, axis=0, keepdims=True)
        ci = cif.astype(jnp.int32) + j * _CH
        keep = bv <= m
        nv = jnp.where(keep, bv, m)
        ni = jnp.where(keep, bi, ci)
        nv = nv.astype(jnp.bfloat16).astype(jnp.float32)
        return nv, ni

    bv0 = jnp.full((1, _TB), jnp.inf, jnp.float32)
    bi0 = jnp.zeros((1, _TB), jnp.int32)
    _, bi = lax.fori_loop(0, _NCH, chunk, (bv0, bi0))
    idx_ref[...] = bi.reshape(1, 1, _TB)


def _argmin_call(cn2, zn3, cbb, zb):
    return pl.pallas_call(
        _argmin_body,
        grid=(8,),
        in_specs=[
            pl.BlockSpec((_CB, 1), lambda i: (0, 0)),
            pl.BlockSpec((1, 1, _TB), lambda i: (i, 0, 0)),
            pl.BlockSpec((_CB, _D), lambda i: (0, 0)),
            pl.BlockSpec((_TB, _D), lambda i: (i, 0)),
        ],
        out_specs=pl.BlockSpec((1, 1, _TB), lambda i: (i, 0, 0)),
        out_shape=jax.ShapeDtypeStruct((8, 1, _TB), jnp.int32),
    )(cn2, zn3, cbb, zb)


def _sc_body(cb_hbm, idx_hbm, zeros_hbm, ones_hbm, zq_hbm, cnt_hbm,
             idx_v0, idx_v1, rows_v0, rows_v1, ones_v, slab_v, counts_sp,
             sem0, sem1):
    c = lax.axis_index("c")
    s = lax.axis_index("s")
    wid = s * _NC + c
    base = wid * _BPW

    pltpu.sync_copy(idx_hbm.at[pl.ds(base, _IDXC)], idx_v0)
    pltpu.sync_copy(idx_hbm.at[pl.ds(base + _IDXC, _IDXC)], idx_v1)
    cp0 = pltpu.async_copy(cb_hbm.at[idx_v0], rows_v0, sem0)
    cp1 = pltpu.async_copy(cb_hbm.at[idx_v1], rows_v1, sem1)

    @pl.when(s == 0)
    def _zero():
        pltpu.sync_copy(zeros_hbm, counts_sp)

    pltpu.sync_copy(ones_hbm, ones_v)
    cp0.wait()
    cp1.wait()
    pltpu.sync_copy(rows_v0, zq_hbm.at[pl.ds(base, _IDXC)])
    pltpu.sync_copy(rows_v1, zq_hbm.at[pl.ds(base + _IDXC, _IDXC)])

    plsc.subcore_barrier()
    pltpu.sync_copy(ones_v, counts_sp.at[idx_v0], add=True)
    pltpu.sync_copy(ones_v, counts_sp.at[idx_v1], add=True)
    plsc.subcore_barrier()

    rows_per_tile = _CB // _NS
    pltpu.sync_copy(counts_sp.at[pl.ds(s * rows_per_tile, rows_per_tile)],
                    slab_v)
    pltpu.sync_copy(slab_v, cnt_hbm.at[c, pl.ds(s * rows_per_tile,
                                                rows_per_tile)])


@functools.partial(
    pl.kernel,
    out_type=(jax.ShapeDtypeStruct((_CB, _D), jnp.float32),
              jax.ShapeDtypeStruct((_NC, _CB, 16), jnp.float32)),
    mesh=plsc.VectorSubcoreMesh(core_axis_name="c", subcore_axis_name="s",
                                num_cores=_NC, num_subcores=_NS),
    compiler_params=pltpu.CompilerParams(use_tc_tiling_on_sc=False),
    scratch_types=(
        pltpu.VMEM((_IDXC,), jnp.int32),
        pltpu.VMEM((_IDXC,), jnp.int32),
        pltpu.VMEM((_IDXC, _D), jnp.float32),
        pltpu.VMEM((_IDXC, _D), jnp.float32),
        pltpu.VMEM((_IDXC, 16), jnp.float32),
        pltpu.VMEM((_CB // _NS, 16), jnp.float32),
        pltpu.VMEM_SHARED((_CB, 16), jnp.float32),
        pltpu.SemaphoreType.DMA,
        pltpu.SemaphoreType.DMA,
    ),
)
def _sc_gather_hist(cb_hbm, idx_hbm, zeros_hbm, ones_hbm, zq_hbm, cnt_hbm,
                    idx_v0, idx_v1, rows_v0, rows_v1, ones_v, slab_v,
                    counts_sp, sem0, sem1):
    _sc_body(cb_hbm, idx_hbm, zeros_hbm, ones_hbm, zq_hbm, cnt_hbm,
             idx_v0, idx_v1, rows_v0, rows_v1, ones_v, slab_v, counts_sp,
             sem0, sem1)


def _epi_body(z_ref, zq_ref, cnt_ref, zqst_ref, loss_ref, perp_ref):
    zv = z_ref[...]
    qv = zq_ref[...]
    diff = qv - zv
    zqst_ref[...] = zv + diff
    msq = jnp.mean(diff * diff)
    loss_ref[...] = jnp.reshape(_BETA * msq + msq, (1, 1))
    c2 = cnt_ref[0] + cnt_ref[1]
    e = c2 * (1.0 / _CB)
    el = e * jnp.log(e + 1e-8)
    ent = jnp.sum(el[:, 0:1])
    perp_ref[...] = jnp.reshape(jnp.exp(-ent), (1, 1))


def _epi_call(z_flat, zq, cnt):
    return pl.pallas_call(
        _epi_body,
        in_specs=[
            pl.BlockSpec((_CB, _D), lambda: (0, 0)),
            pl.BlockSpec((_CB, _D), lambda: (0, 0)),
            pl.BlockSpec((_NC, _CB, 16), lambda: (0, 0, 0)),
        ],
        out_specs=[
            pl.BlockSpec((_CB, _D), lambda: (0, 0)),
            pl.BlockSpec((1, 1), lambda: (0, 0)),
            pl.BlockSpec((1, 1), lambda: (0, 0)),
        ],
        out_shape=[
            jax.ShapeDtypeStruct((_CB, _D), jnp.float32),
            jax.ShapeDtypeStruct((1, 1), jnp.float32),
            jax.ShapeDtypeStruct((1, 1), jnp.float32),
        ],
    )(z_flat, zq, cnt)


def kernel(z, codebook):
    z_flat = z.reshape(-1, _D)
    zb = z_flat.astype(jnp.bfloat16)
    cbb2 = codebook.astype(jnp.bfloat16) * jnp.asarray(2.0, jnp.bfloat16)
    zn = jnp.sum(z_flat ** 2, axis=1)
    cn = jnp.sum(codebook ** 2, axis=1)

    idx = _argmin_call(cn.reshape(_CB, 1), zn.reshape(8, 1, _TB),
                       cbb2, zb).reshape(-1)

    zeros = jnp.zeros((_CB, 16), jnp.float32)
    ones = jnp.ones((_IDXC, 16), jnp.float32)
    z_q, cnt = _sc_gather_hist(codebook, idx, zeros, ones)

    zq_st, loss, perp = _epi_call(z_flat, z_q, cnt)
    return (zq_st.reshape(z.shape), loss[0, 0], perp[0, 0])


# speedup vs baseline: 1.1090x; 1.0641x over previous
import functools

import jax
import jax.numpy as jnp
from jax import lax
from jax.experimental import pallas as pl
from jax.experimental.pallas import tpu as pltpu
from jax.experimental.pallas import tpu_sc as plsc

_CB = 8192
_D = 32
_TB = 1024
_CH = 2048
_NCH = _CB // _CH
_BETA = 0.25

_NC = 2
_NS = 16
_NW = _NC * _NS
_BPW = _CB // _NW
_IDXC = 128


def _argmin_body(cn_ref, zn_ref, cb2_ref, zb_ref, idx_ref):
    zbv = zb_ref[...]
    col = lax.broadcasted_iota(jnp.int32, (8, _CH), 1)
    row = lax.broadcasted_iota(jnp.int32, (8, _CH), 0)
    rsel = jnp.where(
        row == 0, col & 255,
        jnp.where(row == 1, col >> 8,
                  jnp.where(row == 2, 1, 0))).astype(jnp.float32)

    def chunk(j, carry):
        bv, bi = carry
        cbc = cb2_ref[pl.ds(j * _CH, _CH), :]
        mm2 = lax.dot_general(
            cbc, zbv, (((1,), (1,)), ((), ())),
            preferred_element_type=jnp.float32)
        t = cn_ref[pl.ds(j * _CH, _CH), :] + zn_ref[0]
        d = t - mm2
        m = jnp.min(d, axis=0, keepdims=True)
        eqf = jnp.where(d == m, 1.0, 0.0)
        sums = lax.dot_general(
            rsel, eqf, (((1,), (0,)), ((), ())),
            preferred_element_type=jnp.float32)
        cif_fast = sums[1:2] * 256.0 + sums[0:1]
        nmatch = sums[2:3]

        def slow():
            rid = (lax.broadcasted_iota(jnp.int32, (_CH, _TB), 0)
                   .astype(jnp.float32))
            cand = jnp.where(d == m, rid, jnp.inf)
            return jnp.min(cand, axis=0, keepdims=True)

        cif = lax.cond(jnp.max(nmatch) > 1.5, slow, lambda: cif_fast)
        ci = cif.astype(jnp.int32) + j * _CH
        keep = bv <= m
        nv = jnp.where(keep, bv, m)
        ni = jnp.where(keep, bi, ci)
        nv = nv.astype(jnp.bfloat16).astype(jnp.float32)
        return nv, ni

    bv0 = jnp.full((1, _TB), jnp.inf, jnp.float32)
    bi0 = jnp.zeros((1, _TB), jnp.int32)
    _, bi = lax.fori_loop(0, _NCH, chunk, (bv0, bi0))
    idx_ref[...] = bi.reshape(1, 1, _TB)


def _argmin_call(cn2, zn3, cbb, zb):
    return pl.pallas_call(
        _argmin_body,
        grid=(8,),
        in_specs=[
            pl.BlockSpec((_CB, 1), lambda i: (0, 0)),
            pl.BlockSpec((1, 1, _TB), lambda i: (i, 0, 0)),
            pl.BlockSpec((_CB, _D), lambda i: (0, 0)),
            pl.BlockSpec((_TB, _D), lambda i: (i, 0)),
        ],
        out_specs=pl.BlockSpec((1, 1, _TB), lambda i: (i, 0, 0)),
        out_shape=jax.ShapeDtypeStruct((8, 1, _TB), jnp.int32),
    )(cn2, zn3, cbb, zb)


def _sc_body(cb_hbm, idx_hbm, zeros_hbm, ones_hbm, zq_hbm, cnt_hbm,
             idx_v0, idx_v1, rows_v0, rows_v1, ones_v, slab_v, counts_sp,
             sem0, sem1):
    c = lax.axis_index("c")
    s = lax.axis_index("s")
    wid = s * _NC + c
    base = wid * _BPW

    pltpu.sync_copy(idx_hbm.at[pl.ds(base, _IDXC)], idx_v0)
    pltpu.sync_copy(idx_hbm.at[pl.ds(base + _IDXC, _IDXC)], idx_v1)
    cp0 = pltpu.async_copy(cb_hbm.at[idx_v0], rows_v0, sem0)
    cp1 = pltpu.async_copy(cb_hbm.at[idx_v1], rows_v1, sem1)

    @pl.when(s == 0)
    def _zero():
        pltpu.sync_copy(zeros_hbm, counts_sp)

    pltpu.sync_copy(ones_hbm, ones_v)
    cp0.wait()
    cp1.wait()
    pltpu.sync_copy(rows_v0, zq_hbm.at[pl.ds(base, _IDXC)])
    pltpu.sync_copy(rows_v1, zq_hbm.at[pl.ds(base + _IDXC, _IDXC)])

    plsc.subcore_barrier()
    pltpu.sync_copy(ones_v, counts_sp.at[idx_v0], add=True)
    pltpu.sync_copy(ones_v, counts_sp.at[idx_v1], add=True)
    plsc.subcore_barrier()

    rows_per_tile = _CB // _NS
    pltpu.sync_copy(counts_sp.at[pl.ds(s * rows_per_tile, rows_per_tile)],
                    slab_v)
    pltpu.sync_copy(slab_v, cnt_hbm.at[c, pl.ds(s * rows_per_tile,
                                                rows_per_tile)])


@functools.partial(
    pl.kernel,
    out_type=(jax.ShapeDtypeStruct((_CB, _D), jnp.float32),
              jax.ShapeDtypeStruct((_NC, _CB, 16), jnp.float32)),
    mesh=plsc.VectorSubcoreMesh(core_axis_name="c", subcore_axis_name="s",
                                num_cores=_NC, num_subcores=_NS),
    compiler_params=pltpu.CompilerParams(use_tc_tiling_on_sc=False),
    scratch_types=(
        pltpu.VMEM((_IDXC,), jnp.int32),
        pltpu.VMEM((_IDXC,), jnp.int32),
        pltpu.VMEM((_IDXC, _D), jnp.float32),
        pltpu.VMEM((_IDXC, _D), jnp.float32),
        pltpu.VMEM((_IDXC, 16), jnp.float32),
        pltpu.VMEM((_CB // _NS, 16), jnp.float32),
        pltpu.VMEM_SHARED((_CB, 16), jnp.float32),
        pltpu.SemaphoreType.DMA,
        pltpu.SemaphoreType.DMA,
    ),
)
def _sc_gather_hist(cb_hbm, idx_hbm, zeros_hbm, ones_hbm, zq_hbm, cnt_hbm,
                    idx_v0, idx_v1, rows_v0, rows_v1, ones_v, slab_v,
                    counts_sp, sem0, sem1):
    _sc_body(cb_hbm, idx_hbm, zeros_hbm, ones_hbm, zq_hbm, cnt_hbm,
             idx_v0, idx_v1, rows_v0, rows_v1, ones_v, slab_v, counts_sp,
             sem0, sem1)


def _epi_body(z_ref, zq_ref, cnt_ref, zqst_ref, loss_ref, perp_ref):
    zv = z_ref[...]
    qv = zq_ref[...]
    diff = qv - zv
    zqst_ref[...] = zv + diff
    msq = jnp.mean(diff * diff)
    loss_ref[...] = jnp.reshape(_BETA * msq + msq, (1, 1))
    c2 = cnt_ref[0] + cnt_ref[1]
    e = c2 * (1.0 / _CB)
    el = e * jnp.log(e + 1e-8)
    lane = lax.broadcasted_iota(jnp.int32, (_CB // 8, 128), 1)
    ent = jnp.sum(jnp.where((lane & 15) == 0, el, 0.0))
    perp_ref[...] = jnp.reshape(jnp.exp(-ent), (1, 1))


def _epi_call(z_flat, zq, cnt):
    return pl.pallas_call(
        _epi_body,
        in_specs=[
            pl.BlockSpec((_CB, _D), lambda: (0, 0)),
            pl.BlockSpec((_CB, _D), lambda: (0, 0)),
            pl.BlockSpec((_NC, _CB // 8, 128), lambda: (0, 0, 0)),
        ],
        out_specs=[
            pl.BlockSpec((_CB, _D), lambda: (0, 0)),
            pl.BlockSpec((1, 1), lambda: (0, 0)),
            pl.BlockSpec((1, 1), lambda: (0, 0)),
        ],
        out_shape=[
            jax.ShapeDtypeStruct((_CB, _D), jnp.float32),
            jax.ShapeDtypeStruct((1, 1), jnp.float32),
            jax.ShapeDtypeStruct((1, 1), jnp.float32),
        ],
    )(z_flat, zq, cnt)


def kernel(z, codebook):
    z_flat = z.reshape(-1, _D)
    zb = z_flat.astype(jnp.bfloat16)
    cbb2 = codebook.astype(jnp.bfloat16) * jnp.asarray(2.0, jnp.bfloat16)
    zn = jnp.sum(z_flat ** 2, axis=1)
    cn = jnp.sum(codebook ** 2, axis=1)

    idx = _argmin_call(cn.reshape(_CB, 1), zn.reshape(8, 1, _TB),
                       cbb2, zb).reshape(-1)

    zeros = jnp.zeros((_CB, 16), jnp.float32)
    ones = jnp.ones((_IDXC, 16), jnp.float32)
    z_q, cnt = _sc_gather_hist(codebook, idx, zeros, ones)

    zq_st, loss, perp = _epi_call(z_flat, z_q, cnt.reshape(_NC, _CB // 8, 128))
    return (zq_st.reshape(z.shape), loss[0, 0], perp[0, 0])


# speedup vs baseline: 1.1783x; 1.0626x over previous
import functools

import jax
import jax.numpy as jnp
from jax import lax
from jax.experimental import pallas as pl
from jax.experimental.pallas import tpu as pltpu
from jax.experimental.pallas import tpu_sc as plsc

_CB = 8192
_D = 32
_TB = 1024
_CH = 2048
_NCH = _CB // _CH
_BETA = 0.25

_NC = 2
_NS = 16
_NW = _NC * _NS
_BPW = _CB // _NW
_IDXC = 128


def _argmin_body(cn_ref, zn_ref, cb_ref, z_ref, idx_ref):
    zbv = z_ref[...].astype(jnp.bfloat16)
    col = lax.broadcasted_iota(jnp.int32, (8, _CH), 1)
    row = lax.broadcasted_iota(jnp.int32, (8, _CH), 0)
    rsel = jnp.where(
        row == 0, col & 255,
        jnp.where(row == 1, col >> 8,
                  jnp.where(row == 2, 1, 0))).astype(jnp.float32)

    def chunk(j, carry):
        bv, bi = carry
        cbc = (cb_ref[pl.ds(j * _CH, _CH), :] * 2.0).astype(jnp.bfloat16)
        mm2 = lax.dot_general(
            cbc, zbv, (((1,), (1,)), ((), ())),
            preferred_element_type=jnp.float32)
        t = cn_ref[pl.ds(j * _CH, _CH), :] + zn_ref[0]
        d = t - mm2
        m = jnp.min(d, axis=0, keepdims=True)
        eqf = (d == m).astype(jnp.float32)
        sums = lax.dot_general(
            rsel, eqf, (((1,), (0,)), ((), ())),
            preferred_element_type=jnp.float32)
        cif_fast = sums[1:2] * 256.0 + sums[0:1]
        nmatch = sums[2:3]

        def slow():
            rid = (lax.broadcasted_iota(jnp.int32, (_CH, _TB), 0)
                   .astype(jnp.float32))
            cand = jnp.where(d == m, rid, jnp.inf)
            return jnp.min(cand, axis=0, keepdims=True)

        cif = lax.cond(jnp.max(nmatch) > 1.5, slow, lambda: cif_fast)
        ci = cif.astype(jnp.int32) + j * _CH
        keep = bv <= m
        nv = jnp.where(keep, bv, m)
        ni = jnp.where(keep, bi, ci)
        nv = nv.astype(jnp.bfloat16).astype(jnp.float32)
        return nv, ni

    bv0 = jnp.full((1, _TB), jnp.inf, jnp.float32)
    bi0 = jnp.zeros((1, _TB), jnp.int32)
    _, bi = lax.fori_loop(0, _NCH, chunk, (bv0, bi0))
    idx_ref[...] = bi.reshape(1, 1, _TB)


def _argmin_call(cn2, zn3, cb, z_flat):
    return pl.pallas_call(
        _argmin_body,
        grid=(8,),
        in_specs=[
            pl.BlockSpec((_CB, 1), lambda i: (0, 0)),
            pl.BlockSpec((1, 1, _TB), lambda i: (i, 0, 0)),
            pl.BlockSpec((_CB, _D), lambda i: (0, 0)),
            pl.BlockSpec((_TB, _D), lambda i: (i, 0)),
        ],
        out_specs=pl.BlockSpec((1, 1, _TB), lambda i: (i, 0, 0)),
        out_shape=jax.ShapeDtypeStruct((8, 1, _TB), jnp.int32),
    )(cn2, zn3, cb, z_flat)


def _sc_body(cb_hbm, idx_hbm, zeros_hbm, ones_hbm, zq_hbm, cnt_hbm,
             idx_v0, idx_v1, rows_v0, rows_v1, ones_v, slab_v, counts_sp,
             sem0, sem1):
    c = lax.axis_index("c")
    s = lax.axis_index("s")
    wid = s * _NC + c
    base = wid * _BPW

    pltpu.sync_copy(idx_hbm.at[pl.ds(base, _IDXC)], idx_v0)
    pltpu.sync_copy(idx_hbm.at[pl.ds(base + _IDXC, _IDXC)], idx_v1)
    cp0 = pltpu.async_copy(cb_hbm.at[idx_v0], rows_v0, sem0)
    cp1 = pltpu.async_copy(cb_hbm.at[idx_v1], rows_v1, sem1)

    @pl.when(s == 0)
    def _zero():
        pltpu.sync_copy(zeros_hbm, counts_sp)

    pltpu.sync_copy(ones_hbm, ones_v)
    cp0.wait()
    cp1.wait()
    pltpu.sync_copy(rows_v0, zq_hbm.at[pl.ds(base, _IDXC)])
    pltpu.sync_copy(rows_v1, zq_hbm.at[pl.ds(base + _IDXC, _IDXC)])

    plsc.subcore_barrier()
    pltpu.sync_copy(ones_v, counts_sp.at[idx_v0], add=True)
    pltpu.sync_copy(ones_v, counts_sp.at[idx_v1], add=True)
    plsc.subcore_barrier()

    rows_per_tile = _CB // _NS
    pltpu.sync_copy(counts_sp.at[pl.ds(s * rows_per_tile, rows_per_tile)],
                    slab_v)
    pltpu.sync_copy(slab_v, cnt_hbm.at[c, pl.ds(s * rows_per_tile,
                                                rows_per_tile)])


@functools.partial(
    pl.kernel,
    out_type=(jax.ShapeDtypeStruct((_CB, _D), jnp.float32),
              jax.ShapeDtypeStruct((_NC, _CB, 16), jnp.float32)),
    mesh=plsc.VectorSubcoreMesh(core_axis_name="c", subcore_axis_name="s",
                                num_cores=_NC, num_subcores=_NS),
    compiler_params=pltpu.CompilerParams(use_tc_tiling_on_sc=False),
    scratch_types=(
        pltpu.VMEM((_IDXC,), jnp.int32),
        pltpu.VMEM((_IDXC,), jnp.int32),
        pltpu.VMEM((_IDXC, _D), jnp.float32),
        pltpu.VMEM((_IDXC, _D), jnp.float32),
        pltpu.VMEM((_IDXC, 16), jnp.float32),
        pltpu.VMEM((_CB // _NS, 16), jnp.float32),
        pltpu.VMEM_SHARED((_CB, 16), jnp.float32),
        pltpu.SemaphoreType.DMA,
        pltpu.SemaphoreType.DMA,
    ),
)
def _sc_gather_hist(cb_hbm, idx_hbm, zeros_hbm, ones_hbm, zq_hbm, cnt_hbm,
                    idx_v0, idx_v1, rows_v0, rows_v1, ones_v, slab_v,
                    counts_sp, sem0, sem1):
    _sc_body(cb_hbm, idx_hbm, zeros_hbm, ones_hbm, zq_hbm, cnt_hbm,
             idx_v0, idx_v1, rows_v0, rows_v1, ones_v, slab_v, counts_sp,
             sem0, sem1)


def _epi_body(z_ref, zq_ref, cnt_ref, zqst_ref, loss_ref, perp_ref):
    zv = z_ref[...]
    qv = zq_ref[...]
    diff = qv - zv
    zqst_ref[...] = zv + diff
    msq = jnp.mean(diff * diff)
    loss_ref[...] = jnp.reshape(_BETA * msq + msq, (1, 1))
    c2 = cnt_ref[0] + cnt_ref[1]
    e = c2 * (1.0 / _CB)
    el = e * jnp.log(e + 1e-8)
    lane = lax.broadcasted_iota(jnp.int32, (_CB // 8, 128), 1)
    ent = jnp.sum(jnp.where((lane & 15) == 0, el, 0.0))
    perp_ref[...] = jnp.reshape(jnp.exp(-ent), (1, 1))


def _epi_call(z_flat, zq, cnt):
    return pl.pallas_call(
        _epi_body,
        in_specs=[
            pl.BlockSpec((_CB, _D), lambda: (0, 0)),
            pl.BlockSpec((_CB, _D), lambda: (0, 0)),
            pl.BlockSpec((_NC, _CB // 8, 128), lambda: (0, 0, 0)),
        ],
        out_specs=[
            pl.BlockSpec((_CB, _D), lambda: (0, 0)),
            pl.BlockSpec((1, 1), lambda: (0, 0)),
            pl.BlockSpec((1, 1), lambda: (0, 0)),
        ],
        out_shape=[
            jax.ShapeDtypeStruct((_CB, _D), jnp.float32),
            jax.ShapeDtypeStruct((1, 1), jnp.float32),
            jax.ShapeDtypeStruct((1, 1), jnp.float32),
        ],
    )(z_flat, zq, cnt)


def kernel(z, codebook):
    z_flat = z.reshape(-1, _D)
    zn = jnp.sum(z_flat ** 2, axis=1)
    cn = jnp.sum(codebook ** 2, axis=1)

    idx = _argmin_call(cn.reshape(_CB, 1), zn.reshape(8, 1, _TB),
                       codebook, z_flat).reshape(-1)

    zeros = jnp.zeros((_CB, 16), jnp.float32)
    ones = jnp.ones((_IDXC, 16), jnp.float32)
    z_q, cnt = _sc_gather_hist(codebook, idx, zeros, ones)

    zq_st, loss, perp = _epi_call(z_flat, z_q, cnt.reshape(_NC, _CB // 8, 128))
    return (zq_st.reshape(z.shape), loss[0, 0], perp[0, 0])
